# Initial kernel scaffold; baseline (speedup 1.0000x reference)
import jax
import jax.numpy as jnp
from jax.experimental import pallas as pl


def kernel(occ_so, A, B, P):
    raise NotImplementedError("write your pallas kernel here")


# speedup vs baseline: 3.7058x; 3.7058x over previous
import functools

import jax
import jax.numpy as jnp
from jax.experimental import pallas as pl
from jax.experimental.pallas import tpu as pltpu
from jax.experimental.pallas import tpu_sc as plsc

N_SO = 1024
N_E = 64
RANK = 16
BATCH = 4096
TOTAL = BATCH * N_E

GATHER_WINDOW = 128
BB = 256


def _sc_gather(occ2, A, P):
    mesh = plsc.VectorSubcoreMesh(core_axis_name="core", subcore_axis_name="subcore")

    @functools.partial(
        pl.kernel,
        out_type=[jax.ShapeDtypeStruct((TOTAL, RANK), jnp.float32),
                  jax.ShapeDtypeStruct((TOTAL, RANK), jnp.float32)],
        mesh=mesh,
        compiler_params=pltpu.CompilerParams(use_tc_tiling_on_sc=False),
    )
    def k(occ_hbm, a_hbm, p_hbm, oa_hbm, op_hbm):
        def body(i_vmem, oa_vmem, op_vmem):
            pltpu.sync_copy(a_hbm.at[i_vmem.at[0]], oa_vmem)
            pltpu.sync_copy(p_hbm.at[i_vmem.at[0]], op_vmem)

        pltpu.emit_pipeline(
            body,
            grid=(TOTAL // GATHER_WINDOW,),
            in_specs=[pl.BlockSpec((1, GATHER_WINDOW), lambda i: (0, i))],
            out_specs=[pl.BlockSpec((GATHER_WINDOW, RANK), lambda i: (i, 0)),
                       pl.BlockSpec((GATHER_WINDOW, RANK), lambda i: (i, 0))],
            core_axis_name=("core", "subcore"),
            dimension_semantics=(pltpu.PARALLEL,),
        )(occ_hbm, oa_hbm, op_hbm)

    return k(occ2, A, P)


def _tc_contract(ga3, gp3, bt):
    def body(ga_ref, gp_ref, bt_ref, out_ref):
        ga = ga_ref[...]
        w = jnp.mean(gp_ref[...], axis=1)
        s = ga * w[:, None, :]
        out_ref[...] = jnp.dot(s.reshape(BB * N_E, RANK), bt_ref[...],
                               preferred_element_type=jnp.float32)

    return pl.pallas_call(
        body,
        grid=(BATCH // BB,),
        in_specs=[
            pl.BlockSpec((BB, N_E, RANK), lambda i: (i, 0, 0)),
            pl.BlockSpec((BB, N_E, RANK), lambda i: (i, 0, 0)),
            pl.BlockSpec((RANK, N_E), lambda i: (0, 0)),
        ],
        out_specs=pl.BlockSpec((BB * N_E, N_E), lambda i: (i, 0)),
        out_shape=jax.ShapeDtypeStruct((TOTAL, N_E), jnp.float32),
    )(ga3, gp3, bt)


def kernel(occ_so, A, B, P):
    occ2 = occ_so.astype(jnp.int32).reshape(1, TOTAL)
    ga, gp = _sc_gather(occ2, A, P)
    out = _tc_contract(ga.reshape(BATCH, N_E, RANK),
                       gp.reshape(BATCH, N_E, RANK),
                       B.T)
    return out.reshape(BATCH, N_E, N_E)


# speedup vs baseline: 4.3040x; 1.1614x over previous
import functools

import jax
import jax.numpy as jnp
from jax import lax
from jax.experimental import pallas as pl
from jax.experimental.pallas import tpu as pltpu
from jax.experimental.pallas import tpu_sc as plsc

N_SO = 1024
N_E = 64
RANK = 16
BATCH = 4096
TOTAL = BATCH * N_E

NW = 32
CHUNK = TOTAL // NW
W = 2048
BB = 256


def _sc_gather(occ1, A, P):
    mesh = plsc.VectorSubcoreMesh(core_axis_name="core", subcore_axis_name="subcore")

    @functools.partial(
        pl.kernel,
        out_type=[jax.ShapeDtypeStruct((TOTAL, RANK), jnp.float32),
                  jax.ShapeDtypeStruct((TOTAL, RANK), jnp.float32)],
        mesh=mesh,
        compiler_params=pltpu.CompilerParams(use_tc_tiling_on_sc=False),
        scratch_types=[
            pltpu.VMEM((W,), jnp.int32),
            pltpu.VMEM((W, RANK), jnp.float32),
            pltpu.VMEM((W, RANK), jnp.float32),
            pltpu.SemaphoreType.DMA,
            pltpu.SemaphoreType.DMA,
        ],
    )
    def k(occ_hbm, a_hbm, p_hbm, oa_hbm, op_hbm, idx_v, ga_v, gp_v, sem_a, sem_p):
        wid = lax.axis_index("subcore") * 2 + lax.axis_index("core")
        base = wid * CHUNK

        @pl.loop(0, CHUNK // W)
        def _(s):
            off = base + s * W
            pltpu.sync_copy(occ_hbm.at[pl.ds(off, W)], idx_v)
            cp_a = pltpu.async_copy(a_hbm.at[idx_v], ga_v, sem_a)
            cp_p = pltpu.async_copy(p_hbm.at[idx_v], gp_v, sem_p)
            cp_a.wait()
            cp_p.wait()
            pltpu.sync_copy(ga_v, oa_hbm.at[pl.ds(off, W)])
            pltpu.sync_copy(gp_v, op_hbm.at[pl.ds(off, W)])

    return k(occ1, A, P)


def _tc_contract(ga3, gp3, bt):
    def body(ga_ref, gp_ref, bt_ref, out_ref):
        ga = ga_ref[...]
        w = jnp.mean(gp_ref[...], axis=1)
        s = ga * w[:, None, :]
        out_ref[...] = jnp.dot(s.reshape(BB * N_E, RANK), bt_ref[...],
                               preferred_element_type=jnp.float32)

    return pl.pallas_call(
        body,
        grid=(BATCH // BB,),
        in_specs=[
            pl.BlockSpec((BB, N_E, RANK), lambda i: (i, 0, 0)),
            pl.BlockSpec((BB, N_E, RANK), lambda i: (i, 0, 0)),
            pl.BlockSpec((RANK, N_E), lambda i: (0, 0)),
        ],
        out_specs=pl.BlockSpec((BB * N_E, N_E), lambda i: (i, 0)),
        out_shape=jax.ShapeDtypeStruct((TOTAL, N_E), jnp.float32),
    )(ga3, gp3, bt)


def kernel(occ_so, A, B, P):
    occ1 = occ_so.astype(jnp.int32).reshape(TOTAL)
    ga, gp = _sc_gather(occ1, A, P)
    out = _tc_contract(ga.reshape(BATCH, N_E, RANK),
                       gp.reshape(BATCH, N_E, RANK),
                       B.T)
    return out.reshape(BATCH, N_E, N_E)


# speedup vs baseline: 7.1173x; 1.6536x over previous
import functools

import jax
import jax.numpy as jnp
from jax import lax
from jax.experimental import pallas as pl
from jax.experimental.pallas import tpu as pltpu
from jax.experimental.pallas import tpu_sc as plsc

N_SO = 1024
N_E = 64
RANK = 16
BATCH = 4096
TOTAL = BATCH * N_E

NW = 32
CHUNK = TOTAL // NW
W = 2048
BB = 256


def _sc_gather(occ1, A, P):
    mesh = plsc.VectorSubcoreMesh(core_axis_name="core", subcore_axis_name="subcore")

    pack = 128 // RANK

    @functools.partial(
        pl.kernel,
        out_type=[jax.ShapeDtypeStruct((TOTAL, RANK), jnp.float32),
                  jax.ShapeDtypeStruct((TOTAL, RANK), jnp.float32)],
        mesh=mesh,
        compiler_params=pltpu.CompilerParams(use_tc_tiling_on_sc=False),
        scratch_types=[
            pltpu.VMEM((W,), jnp.int32),
            pltpu.VMEM((W, RANK), jnp.float32),
            pltpu.VMEM((W, RANK), jnp.float32),
            pltpu.SemaphoreType.DMA,
            pltpu.SemaphoreType.DMA,
        ],
    )
    def k(occ_hbm, a_hbm, p_hbm, oa_hbm, op_hbm, idx_v, ga_v, gp_v, sem_a, sem_p):
        wid = lax.axis_index("subcore") * 2 + lax.axis_index("core")
        base = wid * CHUNK

        @pl.loop(0, CHUNK // W)
        def _(s):
            off = base + s * W
            pltpu.sync_copy(occ_hbm.at[pl.ds(off, W)], idx_v)
            cp_a = pltpu.async_copy(a_hbm.at[idx_v], ga_v, sem_a)
            cp_p = pltpu.async_copy(p_hbm.at[idx_v], gp_v, sem_p)
            cp_a.wait()
            cp_p.wait()
            pltpu.sync_copy(ga_v, oa_hbm.at[pl.ds(off, W)])
            pltpu.sync_copy(gp_v, op_hbm.at[pl.ds(off, W)])

    return k(occ1, A, P)


def _tc_contract(ga3, gp3, bt):
    pack = 128 // RANK
    rows = BB * N_E // pack

    def body(ga_ref, gp_ref, bd_ref, out_ref):
        ga = ga_ref[...]
        gp = gp_ref[...]
        colsum = jnp.sum(gp.reshape(BB, pack, 128), axis=1)
        w = colsum[:, 0:RANK]
        for c in range(1, pack):
            w = w + colsum[:, c * RANK:(c + 1) * RANK]
        w = w * (1.0 / N_E)
        wt = jnp.concatenate([w] * pack, axis=1)
        wrep = jnp.broadcast_to(wt[:, None, :], (BB, pack, 128)).reshape(rows, 128)
        s = ga * wrep
        o = jnp.dot(s, bd_ref[...], preferred_element_type=jnp.float32)
        o3 = o.reshape(BB, pack, pack * N_E)
        for c in range(pack):
            out_ref[:, pack * c:pack * (c + 1), :] = o3[:, :, N_E * c:N_E * (c + 1)]

    return pl.pallas_call(
        body,
        grid=(BATCH // BB,),
        in_specs=[
            pl.BlockSpec((rows, 128), lambda i: (i, 0)),
            pl.BlockSpec((rows, 128), lambda i: (i, 0)),
            pl.BlockSpec((pack * RANK, pack * N_E), lambda i: (0, 0)),
        ],
        out_specs=pl.BlockSpec((BB, N_E, N_E), lambda i: (i, 0, 0)),
        out_shape=jax.ShapeDtypeStruct((BATCH, N_E, N_E), jnp.float32),
    )(ga3, gp3, bt)


def kernel(occ_so, A, B, P):
    pack = 128 // RANK
    occ1 = (occ_so.astype(jnp.int32)
            .reshape(BATCH, pack, N_E // pack)
            .transpose(0, 2, 1)
            .reshape(TOTAL))
    ga, gp = _sc_gather(occ1, A, P)
    bdiag = jnp.kron(jnp.eye(pack, dtype=jnp.float32), B.T)
    return _tc_contract(ga.reshape(TOTAL // pack, 128),
                        gp.reshape(TOTAL // pack, 128),
                        bdiag)


# speedup vs baseline: 7.3158x; 1.0279x over previous
import functools

import jax
import jax.numpy as jnp
from jax import lax
from jax.experimental import pallas as pl
from jax.experimental.pallas import tpu as pltpu
from jax.experimental.pallas import tpu_sc as plsc

N_SO = 1024
N_E = 64
RANK = 16
BATCH = 4096
TOTAL = BATCH * N_E

PACK = 128 // (2 * RANK)
NW = 32
CHUNK = TOTAL // NW
W = 2048
WC = W // PACK
BB = 256


def _sc_gather(occ2, AP):
    mesh = plsc.VectorSubcoreMesh(core_axis_name="core", subcore_axis_name="subcore")

    @functools.partial(
        pl.kernel,
        out_type=jax.ShapeDtypeStruct((TOTAL // PACK, 128), jnp.float32),
        mesh=mesh,
        compiler_params=pltpu.CompilerParams(use_tc_tiling_on_sc=False),
        scratch_types=[
            pltpu.VMEM((WC,), jnp.int32),
            pltpu.VMEM((WC,), jnp.int32),
            pltpu.VMEM((WC,), jnp.int32),
            pltpu.VMEM((WC,), jnp.int32),
            pltpu.VMEM((WC, 2 * RANK), jnp.float32),
            pltpu.VMEM((WC, 2 * RANK), jnp.float32),
            pltpu.VMEM((WC, 2 * RANK), jnp.float32),
            pltpu.VMEM((WC, 2 * RANK), jnp.float32),
            pltpu.SemaphoreType.DMA,
            pltpu.SemaphoreType.DMA,
        ],
    )
    def k(occ_hbm, ap_hbm, o_hbm, i0, i1, i2, i3, g0, g1, g2, g3, sem, osem):
        wid = lax.axis_index("subcore") * 2 + lax.axis_index("core")
        base = wid * CHUNK
        idx_refs = [i0, i1, i2, i3]
        g_refs = [g0, g1, g2, g3]

        @pl.loop(0, CHUNK // W)
        def _(s):
            off = base + s * W
            for c in range(PACK):
                pltpu.sync_copy(occ_hbm.at[pl.ds(off + c * WC, WC)], idx_refs[c])
            cps = [
                pltpu.async_copy(ap_hbm.at[idx_refs[c]], g_refs[c], sem)
                for c in range(PACK)
            ]
            for cp in cps:
                cp.wait()
            ocps = [
                pltpu.async_copy(
                    g_refs[c],
                    o_hbm.at[pl.ds(off // PACK, W // PACK),
                             pl.ds(2 * RANK * c, 2 * RANK)],
                    osem,
                )
                for c in range(PACK)
            ]
            for cp in ocps:
                cp.wait()

    return k(occ2, AP)


def _tc_contract(gap, bdiag):
    rows = BB * N_E // PACK
    grp = N_E // PACK

    def body(gap_ref, bd_ref, out_ref):
        g = gap_ref[...]
        colsum = jnp.sum(g.reshape(BB, grp, 128), axis=1)
        w = colsum[:, RANK:2 * RANK]
        for c in range(1, PACK):
            w = w + colsum[:, 2 * RANK * c + RANK:2 * RANK * (c + 1)]
        w = w * (1.0 / N_E)
        wt = jnp.concatenate([w] * (128 // RANK), axis=1)
        wrep = jnp.broadcast_to(wt[:, None, :], (BB, grp, 128)).reshape(rows, 128)
        s = g * wrep
        o = jnp.dot(s, bd_ref[...], preferred_element_type=jnp.float32)
        o3 = o.reshape(BB, grp, PACK * N_E)
        for c in range(PACK):
            out_ref[:, grp * c:grp * (c + 1), :] = o3[:, :, N_E * c:N_E * (c + 1)]

    return pl.pallas_call(
        body,
        grid=(BATCH // BB,),
        in_specs=[
            pl.BlockSpec((rows, 128), lambda i: (i, 0)),
            pl.BlockSpec((128, PACK * N_E), lambda i: (0, 0)),
        ],
        out_specs=pl.BlockSpec((BB, N_E, N_E), lambda i: (i, 0, 0)),
        out_shape=jax.ShapeDtypeStruct((BATCH, N_E, N_E), jnp.float32),
    )(gap, bdiag)


def kernel(occ_so, A, B, P):
    occ1 = (occ_so.astype(jnp.int32)
            .reshape(BATCH, PACK, N_E // PACK)
            .transpose(0, 2, 1)
            .reshape(TOTAL))
    occ2 = (occ1.reshape(TOTAL // W, WC, PACK)
            .swapaxes(1, 2)
            .reshape(TOTAL))
    AP = jnp.concatenate([A, P], axis=1)
    gap = _sc_gather(occ2, AP)
    bt0 = jnp.concatenate([B.T, jnp.zeros((RANK, N_E), jnp.float32)], axis=0)
    bdiag = jnp.kron(jnp.eye(PACK, dtype=jnp.float32), bt0)
    return _tc_contract(gap, bdiag)


# speedup vs baseline: 10.2940x; 1.4071x over previous
import functools

import jax
import jax.numpy as jnp
from jax import lax
from jax.experimental import pallas as pl
from jax.experimental.pallas import tpu as pltpu
from jax.experimental.pallas import tpu_sc as plsc

N_SO = 1024
N_E = 64
RANK = 16
BATCH = 4096
TOTAL = BATCH * N_E

PACK = 128 // (2 * RANK)
NW = 32
CHUNK = TOTAL // NW
W = 2048
WC = W // PACK
BB = 256


def _sc_gather(occ2, AP):
    mesh = plsc.VectorSubcoreMesh(core_axis_name="core", subcore_axis_name="subcore")

    @functools.partial(
        pl.kernel,
        out_type=jax.ShapeDtypeStruct((TOTAL // PACK, 128), jnp.float32),
        mesh=mesh,
        compiler_params=pltpu.CompilerParams(use_tc_tiling_on_sc=False),
        scratch_types=[
            pltpu.VMEM((WC,), jnp.int32),
            pltpu.VMEM((WC,), jnp.int32),
            pltpu.VMEM((WC,), jnp.int32),
            pltpu.VMEM((WC,), jnp.int32),
            pltpu.VMEM((WC, 2 * RANK), jnp.float32),
            pltpu.VMEM((WC, 2 * RANK), jnp.float32),
            pltpu.VMEM((WC, 2 * RANK), jnp.float32),
            pltpu.VMEM((WC, 2 * RANK), jnp.float32),
            pltpu.SemaphoreType.DMA,
            pltpu.SemaphoreType.DMA,
        ],
    )
    def k(occ_hbm, ap_hbm, o_hbm, i0, i1, i2, i3, g0, g1, g2, g3, sem, osem):
        wid = lax.axis_index("subcore") * 2 + lax.axis_index("core")
        base = wid * CHUNK
        idx_refs = [i0, i1, i2, i3]
        g_refs = [g0, g1, g2, g3]

        @pl.loop(0, CHUNK // W)
        def _(s):
            off = base + s * W
            for c in range(PACK):
                pltpu.sync_copy(occ_hbm.at[pl.ds(off + c * WC, WC)], idx_refs[c])
            cps = [
                pltpu.async_copy(ap_hbm.at[idx_refs[c]], g_refs[c], sem)
                for c in range(PACK)
            ]
            for cp in cps:
                cp.wait()
            ocps = [
                pltpu.async_copy(
                    g_refs[c],
                    o_hbm.at[pl.ds(off // PACK, W // PACK),
                             pl.ds(2 * RANK * c, 2 * RANK)],
                    osem,
                )
                for c in range(PACK)
            ]
            for cp in ocps:
                cp.wait()

    return k(occ2, AP)


def _tc_contract(gap, bdiag, ident):
    rows = BB * N_E // PACK
    grp = N_E // PACK

    def body(gap_ref, bd_ref, id_ref, out_ref):
        g = gap_ref[...]
        colsum = jnp.sum(g.reshape(BB, grp, 128), axis=1)
        w = colsum[:, RANK:2 * RANK]
        for c in range(1, PACK):
            w = w + colsum[:, 2 * RANK * c + RANK:2 * RANK * (c + 1)]
        w = w * (1.0 / N_E)
        wt = jnp.concatenate([w] * (128 // RANK), axis=1)
        wrep = jnp.broadcast_to(wt[:, None, :], (BB, grp, 128)).reshape(rows, 128)
        s = g * wrep
        o = jnp.dot(s, bd_ref[...], preferred_element_type=jnp.float32)
        o3 = o.reshape(BB, grp, PACK * N_E)
        ident_b = id_ref[...]
        for c in range(PACK):
            for j in range(grp):
                ot = jax.lax.dot_general(
                    o3[:, j, N_E * c:N_E * (c + 1)], ident_b,
                    (((0,), (0,)), ((), ())),
                    preferred_element_type=jnp.float32)
                out_ref[grp * c + j, :, :] = ot

    return pl.pallas_call(
        body,
        grid=(BATCH // BB,),
        in_specs=[
            pl.BlockSpec((rows, 128), lambda i: (i, 0)),
            pl.BlockSpec((128, PACK * N_E), lambda i: (0, 0)),
            pl.BlockSpec((BB, BB), lambda i: (0, 0)),
        ],
        out_specs=pl.BlockSpec((N_E, N_E, BB), lambda i: (0, 0, i)),
        out_shape=jax.ShapeDtypeStruct((N_E, N_E, BATCH), jnp.float32),
    )(gap, bdiag, ident)


def kernel(occ_so, A, B, P):
    occ1 = (occ_so.astype(jnp.int32)
            .reshape(BATCH, PACK, N_E // PACK)
            .transpose(0, 2, 1)
            .reshape(TOTAL))
    occ2 = (occ1.reshape(TOTAL // W, WC, PACK)
            .swapaxes(1, 2)
            .reshape(TOTAL))
    AP = jnp.concatenate([A, P], axis=1)
    gap = _sc_gather(occ2, AP)
    bt0 = jnp.concatenate([B.T, jnp.zeros((RANK, N_E), jnp.float32)], axis=0)
    bdiag = jnp.kron(jnp.eye(PACK, dtype=jnp.float32), bt0)
    ident = jnp.eye(BB, dtype=jnp.float32)
    out_t = _tc_contract(gap, bdiag, ident)
    return jnp.transpose(out_t, (2, 0, 1))


# speedup vs baseline: 11.0184x; 1.0704x over previous
import functools

import jax
import jax.numpy as jnp
from jax import lax
from jax.experimental import pallas as pl
from jax.experimental.pallas import tpu as pltpu
from jax.experimental.pallas import tpu_sc as plsc

N_SO = 1024
N_E = 64
RANK = 16
BATCH = 4096
TOTAL = BATCH * N_E

PACK = 128 // (2 * RANK)
NW = 32
CHUNK = TOTAL // NW
W = 2048
WC = W // PACK
BB = 256


def _sc_gather(occ2, AP):
    mesh = plsc.VectorSubcoreMesh(core_axis_name="core", subcore_axis_name="subcore")

    @functools.partial(
        pl.kernel,
        out_type=jax.ShapeDtypeStruct((TOTAL // PACK, 128), jnp.float32),
        mesh=mesh,
        compiler_params=pltpu.CompilerParams(use_tc_tiling_on_sc=False),
        scratch_types=[
            pltpu.VMEM((WC,), jnp.int32),
            pltpu.VMEM((WC,), jnp.int32),
            pltpu.VMEM((WC,), jnp.int32),
            pltpu.VMEM((WC,), jnp.int32),
            pltpu.VMEM((WC, 2 * RANK), jnp.float32),
            pltpu.VMEM((WC, 2 * RANK), jnp.float32),
            pltpu.VMEM((WC, 2 * RANK), jnp.float32),
            pltpu.VMEM((WC, 2 * RANK), jnp.float32),
            pltpu.SemaphoreType.DMA,
            pltpu.SemaphoreType.DMA,
        ],
    )
    def k(occ_hbm, ap_hbm, o_hbm, i0, i1, i2, i3, g0, g1, g2, g3, sem, osem):
        wid = lax.axis_index("subcore") * 2 + lax.axis_index("core")
        base = wid * CHUNK
        idx_refs = [i0, i1, i2, i3]
        g_refs = [g0, g1, g2, g3]

        @pl.loop(0, CHUNK // W)
        def _(s):
            off = base + s * W
            for c in range(PACK):
                pltpu.sync_copy(occ_hbm.at[pl.ds(off + c * WC, WC)], idx_refs[c])
            cps = [
                pltpu.async_copy(ap_hbm.at[idx_refs[c]], g_refs[c], sem)
                for c in range(PACK)
            ]
            for cp in cps:
                cp.wait()
            ocps = [
                pltpu.async_copy(
                    g_refs[c],
                    o_hbm.at[pl.ds(off // PACK, W // PACK),
                             pl.ds(2 * RANK * c, 2 * RANK)],
                    osem,
                )
                for c in range(PACK)
            ]
            for cp in ocps:
                cp.wait()

    return k(occ2, AP)


def _tc_contract(gap, bdiag, ident):
    rows = BB * N_E // PACK
    grp = N_E // PACK

    def body(gap_ref, bd_ref, id_ref, out_ref):
        g = gap_ref[...]
        colsum = jnp.sum(g.reshape(BB, grp, 128), axis=1)
        w = colsum[:, RANK:2 * RANK]
        for c in range(1, PACK):
            w = w + colsum[:, 2 * RANK * c + RANK:2 * RANK * (c + 1)]
        w = w * (1.0 / N_E)
        wt = jnp.concatenate([w] * (128 // RANK), axis=1)
        wrep = jnp.broadcast_to(wt[:, None, :], (BB, grp, 128)).reshape(rows, 128)
        s = g * wrep
        o = jnp.dot(s, bd_ref[...], preferred_element_type=jnp.float32)
        o3 = o.reshape(BB, grp, PACK * N_E)
        ident_b = id_ref[...]
        for j in range(grp):
            ot = jax.lax.dot_general(
                o3[:, j, :], ident_b,
                (((0,), (0,)), ((), ())),
                preferred_element_type=jnp.float32)
            for c in range(PACK):
                out_ref[grp * c + j, :, :] = ot[N_E * c:N_E * (c + 1), :]

    return pl.pallas_call(
        body,
        grid=(BATCH // BB,),
        in_specs=[
            pl.BlockSpec((rows, 128), lambda i: (i, 0)),
            pl.BlockSpec((128, PACK * N_E), lambda i: (0, 0)),
            pl.BlockSpec((BB, BB), lambda i: (0, 0)),
        ],
        out_specs=pl.BlockSpec((N_E, N_E, BB), lambda i: (0, 0, i)),
        out_shape=jax.ShapeDtypeStruct((N_E, N_E, BATCH), jnp.float32),
    )(gap, bdiag, ident)


def kernel(occ_so, A, B, P):
    occ1 = (occ_so.astype(jnp.int32)
            .reshape(BATCH, PACK, N_E // PACK)
            .transpose(0, 2, 1)
            .reshape(TOTAL))
    occ2 = (occ1.reshape(TOTAL // W, WC, PACK)
            .swapaxes(1, 2)
            .reshape(TOTAL))
    AP = jnp.concatenate([A, P], axis=1)
    gap = _sc_gather(occ2, AP)
    bt0 = jnp.concatenate([B.T, jnp.zeros((RANK, N_E), jnp.float32)], axis=0)
    bdiag = jnp.kron(jnp.eye(PACK, dtype=jnp.float32), bt0)
    ident = jnp.eye(BB, dtype=jnp.float32)
    out_t = _tc_contract(gap, bdiag, ident)
    return jnp.transpose(out_t, (2, 0, 1))
